# speedup vs baseline: 1.0658x; 1.0658x over previous
import jax
import jax.numpy as jnp
from jax.experimental import pallas as pl
from jax.experimental.pallas import tpu as pltpu

E = 8
K = 2
D = 768
F = 3072
FB = 512


def _router_body(x_ref, cw_ref, cb_ref, w_ref, sel_ref, rw_ref, pay_ref,
                 coef_ref):
    x = x_ref[...]
    cw = cw_ref[...]
    logits = jax.lax.dot_general(
        x, cw, (((1,), (1,)), ((), ())), preferred_element_type=jnp.float32)
    logits = logits + cb_ref[...]
    bids = (1.0 / (1.0 + jnp.exp(-logits))) * w_ref[...]
    s = bids.shape[0]
    neg_inf = jnp.float32(float('-inf'))
    ei = jax.lax.broadcasted_iota(jnp.int32, (s, E), 1)
    max1 = jnp.max(bids, axis=1, keepdims=True)
    idx1 = jnp.min(jnp.where(bids == max1, ei, E), axis=1, keepdims=True)
    b2 = jnp.where(ei == idx1, neg_inf, bids)
    max2 = jnp.max(b2, axis=1, keepdims=True)
    idx2 = jnp.min(jnp.where(b2 == max2, ei, E), axis=1, keepdims=True)
    b3 = jnp.where(ei == idx2, neg_inf, b2)
    max3 = jnp.max(b3, axis=1, keepdims=True)
    t = jnp.exp(max2 - max1)
    rw1 = 1.0 / (1.0 + t)
    rw2 = t / (1.0 + t)
    sel_ref[...] = jnp.concatenate([idx1, idx2], axis=1)
    rw_ref[...] = jnp.concatenate([rw1, rw2], axis=1)
    pay_ref[...] = jnp.concatenate([max3, max3], axis=1)
    coef_ref[...] = (rw1 * (ei == idx1).astype(jnp.float32)
                     + rw2 * (ei == idx2).astype(jnp.float32))


def _ffn_body(x_ref, gw_ref, uw_ref, dw_ref, c_ref, out_ref, acc_ref):
    e = pl.program_id(0)
    f = pl.program_id(1)

    @pl.when((e == 0) & (f == 0))
    def _init():
        acc_ref[...] = jnp.zeros_like(acc_ref)

    x = x_ref[...]
    g = jax.lax.dot(x, gw_ref[0], preferred_element_type=jnp.float32)
    u = jax.lax.dot(x, uw_ref[0], preferred_element_type=jnp.float32)
    h = (g / (1.0 + jnp.exp(-g))) * u
    h = h * c_ref[0]
    acc_ref[...] += jax.lax.dot(
        h.astype(jnp.bfloat16), dw_ref[0], preferred_element_type=jnp.float32)

    @pl.when((e == E - 1) & (f == F // FB - 1))
    def _fin():
        out_ref[...] = acc_ref[...]


def kernel(hidden_states, conf_w, conf_b, gate_w, up_w, down_w, wealth):
    b, s, d = hidden_states.shape
    x = hidden_states.reshape(s, d)

    sel, rw, pay, coef = pl.pallas_call(
        _router_body,
        out_shape=(
            jax.ShapeDtypeStruct((s, K), jnp.int32),
            jax.ShapeDtypeStruct((s, K), jnp.float32),
            jax.ShapeDtypeStruct((s, K), jnp.float32),
            jax.ShapeDtypeStruct((s, E), jnp.float32),
        ),
    )(x, conf_w, conf_b.reshape(1, E), wealth.reshape(1, E))

    coef_t = coef.T.reshape(E, s, 1)
    xb = x.astype(jnp.bfloat16)
    gwb = gate_w.astype(jnp.bfloat16)
    uwb = up_w.astype(jnp.bfloat16)
    dwb = down_w.astype(jnp.bfloat16)

    out = pl.pallas_call(
        _ffn_body,
        grid=(E, F // FB),
        in_specs=[
            pl.BlockSpec((s, D), lambda e, f: (0, 0)),
            pl.BlockSpec((1, D, FB), lambda e, f: (e, 0, f)),
            pl.BlockSpec((1, D, FB), lambda e, f: (e, 0, f)),
            pl.BlockSpec((1, FB, D), lambda e, f: (e, f, 0)),
            pl.BlockSpec((1, s, 1), lambda e, f: (e, 0, 0)),
        ],
        out_specs=pl.BlockSpec((s, D), lambda e, f: (0, 0)),
        out_shape=jax.ShapeDtypeStruct((s, D), jnp.float32),
        scratch_shapes=[pltpu.VMEM((s, D), jnp.float32)],
    )(xb, gwb, uwb, dwb, coef_t)

    return (out.reshape(b, s, d), sel.reshape(b, s, K),
            rw.reshape(b, s, K), pay.reshape(b, s, K))


# speedup vs baseline: 1.4268x; 1.3387x over previous
import jax
import jax.numpy as jnp
from jax.experimental import pallas as pl
from jax.experimental.pallas import tpu as pltpu

E = 8
K = 2
D = 768
F = 3072
FB = 512


def _router_body(x_ref, cw_ref, cb_ref, w_ref, sel_ref, rw_ref, pay_ref,
                 coef_ref):
    x = x_ref[...]
    cw = cw_ref[...]
    logits = jax.lax.dot_general(
        x, cw, (((1,), (1,)), ((), ())), preferred_element_type=jnp.float32)
    logits = logits + cb_ref[...]
    bids = (1.0 / (1.0 + jnp.exp(-logits))) * w_ref[...]
    s = bids.shape[0]
    neg_inf = jnp.float32(float('-inf'))
    ei = jax.lax.broadcasted_iota(jnp.int32, (s, E), 1)
    max1 = jnp.max(bids, axis=1, keepdims=True)
    idx1 = jnp.min(jnp.where(bids == max1, ei, E), axis=1, keepdims=True)
    b2 = jnp.where(ei == idx1, neg_inf, bids)
    max2 = jnp.max(b2, axis=1, keepdims=True)
    idx2 = jnp.min(jnp.where(b2 == max2, ei, E), axis=1, keepdims=True)
    b3 = jnp.where(ei == idx2, neg_inf, b2)
    max3 = jnp.max(b3, axis=1, keepdims=True)
    t = jnp.exp(max2 - max1)
    rw1 = 1.0 / (1.0 + t)
    rw2 = t / (1.0 + t)
    sel_ref[...] = jnp.concatenate([idx1, idx2], axis=1)
    rw_ref[...] = jnp.concatenate([rw1, rw2], axis=1)
    pay_ref[...] = jnp.concatenate([max3, max3], axis=1)
    coef_ref[...] = (rw1 * (ei == idx1).astype(jnp.float32)
                     + rw2 * (ei == idx2).astype(jnp.float32))


def _ffn_body(x_ref, gw_ref, uw_ref, dw_ref, c_ref, out_ref, acc_ref):
    e = pl.program_id(0)
    f = pl.program_id(1)

    @pl.when((e == 0) & (f == 0))
    def _init():
        acc_ref[...] = jnp.zeros_like(acc_ref)

    x = x_ref[...]
    gw = gw_ref[0].astype(jnp.bfloat16)
    uw = uw_ref[0].astype(jnp.bfloat16)
    dw = dw_ref[0].astype(jnp.bfloat16)
    g = jax.lax.dot(x, gw, preferred_element_type=jnp.float32)
    u = jax.lax.dot(x, uw, preferred_element_type=jnp.float32)
    h = (g / (1.0 + jnp.exp(-g))) * u
    h = h * c_ref[0]
    acc_ref[...] += jax.lax.dot(
        h.astype(jnp.bfloat16), dw, preferred_element_type=jnp.float32)

    @pl.when((e == E - 1) & (f == F // FB - 1))
    def _fin():
        out_ref[...] = acc_ref[...]


def kernel(hidden_states, conf_w, conf_b, gate_w, up_w, down_w, wealth):
    b, s, d = hidden_states.shape
    x = hidden_states.reshape(s, d)

    sel, rw, pay, coef = pl.pallas_call(
        _router_body,
        out_shape=(
            jax.ShapeDtypeStruct((s, K), jnp.int32),
            jax.ShapeDtypeStruct((s, K), jnp.float32),
            jax.ShapeDtypeStruct((s, K), jnp.float32),
            jax.ShapeDtypeStruct((s, E), jnp.float32),
        ),
    )(x, conf_w, conf_b.reshape(1, E), wealth.reshape(1, E))

    coef_t = coef.T.reshape(E, s, 1)
    xb = x.astype(jnp.bfloat16)

    out = pl.pallas_call(
        _ffn_body,
        grid=(E, F // FB),
        in_specs=[
            pl.BlockSpec((s, D), lambda e, f: (0, 0)),
            pl.BlockSpec((1, D, FB), lambda e, f: (e, 0, f)),
            pl.BlockSpec((1, D, FB), lambda e, f: (e, 0, f)),
            pl.BlockSpec((1, FB, D), lambda e, f: (e, f, 0)),
            pl.BlockSpec((1, s, 1), lambda e, f: (e, 0, 0)),
        ],
        out_specs=pl.BlockSpec((s, D), lambda e, f: (0, 0)),
        out_shape=jax.ShapeDtypeStruct((s, D), jnp.float32),
        scratch_shapes=[pltpu.VMEM((s, D), jnp.float32)],
    )(xb, gate_w, up_w, down_w, coef_t)

    return (out.reshape(b, s, d), sel.reshape(b, s, K),
            rw.reshape(b, s, K), pay.reshape(b, s, K))
